# baseline (speedup 1.0000x reference)
import functools

import jax
import jax.numpy as jnp
from jax.experimental import pallas as pl
from jax.experimental.pallas import tpu as pltpu
from jax.experimental.pallas import tpu_sc as plsc

B = 32
T = 64
F = 128
H = 256
M = 5
OUT = 128
E_T = 512
E_S = 128
XIN = H + 2 * F
G = 8

_F32 = jnp.float32
_BF16 = jnp.bfloat16


def _dot(a, b):
    return jax.lax.dot_general(a, b, (((1,), (0,)), ((), ())),
                               preferred_element_type=_F32)


def _dot_t(a, b):
    return jax.lax.dot_general(a, b, (((1,), (1,)), ((), ())),
                               preferred_element_type=_F32)


def _dot_lt(a, b):
    return jax.lax.dot_general(a, b, (((0,), (0,)), ((), ())),
                               preferred_element_type=_F32)


def _b16(x):
    return x.astype(_BF16)


_LN = 16


def _edge_mats_sc_body(src_hbm, dst_hbm, wgt_hbm, af_hbm, ab_hbm,
                       src_v, dst_v, w_v, dego_v, degi_v, afv, abv):
    c = jax.lax.axis_index("c")
    s = jax.lax.axis_index("s")
    wid = s * 2 + c
    pltpu.sync_copy(src_hbm.at[wid], src_v)
    pltpu.sync_copy(dst_hbm.at[wid], dst_v)
    pltpu.sync_copy(wgt_hbm.at[wid], w_v)
    zf = jnp.zeros((_LN,), _F32)
    for i in range(T // _LN):
        dego_v[pl.ds(i * _LN, _LN)] = zf
        degi_v[pl.ds(i * _LN, _LN)] = zf
    for i in range(E_T // _LN):
        sl = pl.ds(i * _LN, _LN)
        plsc.addupdate_scatter(dego_v, [src_v[sl]], w_v[sl])
        plsc.addupdate_scatter(degi_v, [dst_v[sl]], w_v[sl])
    for i in range(T * T // _LN):
        sl = pl.ds(i * _LN, _LN)
        afv[sl] = zf
        abv[sl] = zf
    for i in range(T // _LN):
        sl = pl.ds(i * _LN, _LN)
        d = dego_v[sl]
        dego_v[sl] = jnp.where(d > 0, d, 1.0)
        d = degi_v[sl]
        degi_v[sl] = jnp.where(d > 0, d, 1.0)
    for i in range(E_T // _LN):
        sl = pl.ds(i * _LN, _LN)
        sv = src_v[sl]
        dv = dst_v[sl]
        wv = w_v[sl]
        wf = wv / plsc.load_gather(dego_v, [sv])
        wb = wv / plsc.load_gather(degi_v, [dv])
        plsc.addupdate_scatter(afv, [dv * T + sv], wf)
        plsc.addupdate_scatter(abv, [sv * T + dv], wb)
    pltpu.sync_copy(afv, af_hbm.at[wid])
    pltpu.sync_copy(abv, ab_hbm.at[wid])


def _edge_mats_sc(srcs, dsts, wgts):
    k = functools.partial(
        pl.kernel,
        out_type=[jax.ShapeDtypeStruct((B, T * T), _F32),
                  jax.ShapeDtypeStruct((B, T * T), _F32)],
        mesh=plsc.VectorSubcoreMesh(core_axis_name="c", subcore_axis_name="s"),
        compiler_params=pltpu.CompilerParams(needs_layout_passes=False),
        scratch_types=[pltpu.VMEM((E_T,), jnp.int32),
                       pltpu.VMEM((E_T,), jnp.int32),
                       pltpu.VMEM((E_T,), _F32),
                       pltpu.VMEM((T,), _F32),
                       pltpu.VMEM((T,), _F32),
                       pltpu.VMEM((T * T,), _F32),
                       pltpu.VMEM((T * T,), _F32)],
    )(_edge_mats_sc_body)
    return k(srcs, dsts, wgts)


def _graph_kernel(x_ref, af_ref, ab_ref, ei_ref, ew_ref,
                  wt_ref, bt_ref, ws_ref, bs_ref,
                  xin_ref, afs_ref, abs_ref):
    b = pl.program_id(0)

    @pl.when(b == 0)
    def _():
        src = ei_ref[0:1, :].astype(jnp.int32)
        dst = ei_ref[1:2, :].astype(jnp.int32)
        w = ew_ref[...]
        iota = jax.lax.broadcasted_iota(jnp.int32, (F, E_S), 0)
        gs = (iota == src).astype(_F32)
        gd = (iota == dst).astype(_F32)
        deg_out = jnp.sum(gs * w, axis=1, keepdims=True)
        deg_in = jnp.sum(gd * w, axis=1, keepdims=True)
        dso = jnp.where(deg_out > 0, deg_out, 1.0)
        dsi = jnp.where(deg_in > 0, deg_in, 1.0)
        w_fwd = w / jnp.sum(gs * dso, axis=0, keepdims=True)
        w_bwd = w / jnp.sum(gd * dsi, axis=0, keepdims=True)
        afs_ref[...] = _b16(_dot_t(_b16(gs), _b16(gd * w_fwd)))
        abs_ref[...] = _b16(_dot_t(_b16(gd), _b16(gs * w_bwd)))

    afs = afs_ref[...]
    abs_ = abs_ref[...]

    for j in range(G):
        a_f = _b16(af_ref[j])
        a_b = _b16(ab_ref[j])

        xb = _b16(x_ref[j])
        zf1 = _b16(_dot(a_f, xb))
        zf2 = _b16(_dot(a_f, zf1))
        zb1 = _b16(_dot(a_b, xb))
        zb2 = _b16(_dot(a_b, zb1))
        dt = (_dot(zf1, wt_ref[0:F]) + _dot(zf2, wt_ref[F:2 * F])
              + _dot(zb1, wt_ref[2 * F:3 * F]) + _dot(zb2, wt_ref[3 * F:4 * F])
              + bt_ref[...])

        y1 = _b16(_dot(xb, afs))
        y2 = _b16(_dot(y1, afs))
        y3 = _b16(_dot(xb, abs_))
        y4 = _b16(_dot(y3, abs_))
        ds = (_dot(ws_ref[0], y1) + _dot(ws_ref[1], y2)
              + _dot(ws_ref[2], y3) + _dot(ws_ref[3], y4)
              + bs_ref[...])

        xin_ref[j] = jnp.concatenate([_b16(dt), _b16(ds), xb], axis=1)


def _proj_lstm_head_kernel(xin_ref, wih_ref, bg_ref, whh_ref,
                           muw_ref, mub_ref, sgw_ref, sgb_ref,
                           piw_ref, pib_ref,
                           mu_ref, sg_ref, pi_ref, p_ref, hs_ref):
    xin = xin_ref[...].reshape(T * B, XIN)
    p_ref[...] = (_dot(xin, wih_ref[...]) + bg_ref[...]).reshape(T, B, 4 * H)

    def _sig(x):
        return 0.5 * jnp.tanh(0.5 * x) + 0.5

    def step(t, carry):
        h, c = carry
        pt = p_ref[pl.ds(t, 1)].reshape(B, 4 * H)
        gates = pt + _dot(h, whh_ref[...])
        i = _sig(gates[:, 0:H])
        f = _sig(gates[:, H:2 * H])
        g = jnp.tanh(gates[:, 2 * H:3 * H])
        o = _sig(gates[:, 3 * H:4 * H])
        c2 = f * c + i * g
        h2 = o * jnp.tanh(c2)
        hs_ref[pl.ds(t, 1)] = h2.reshape(1, B, H)
        return (_b16(h2), c2)

    zeros16 = jnp.zeros((B, H), _BF16)
    zeros = jnp.zeros((B, H), _F32)
    jax.lax.fori_loop(0, T, step, (zeros16, zeros))

    hs = _b16(jnp.swapaxes(hs_ref[...], 0, 1).reshape(B * T, H))
    mu_ref[...] = _dot(hs, muw_ref[...]) + mub_ref[...]
    sg_ref[...] = jnp.exp(_dot(hs, sgw_ref[...]) + sgb_ref[...])
    logits = _dot(hs, piw_ref[...]) + pib_ref[...]
    mx = jnp.max(logits, axis=-1, keepdims=True)
    e = jnp.exp(logits - mx)
    pi_ref[...] = e / jnp.sum(e, axis=-1, keepdims=True)


def kernel(x, temporal_edge_i, temporal_edge_w, edge_index, edge_weight,
           Wt, bt, Ws, bs, W_ih, W_hh, b_ih, b_hh,
           mu_w, mu_b, sigma_w, sigma_b, pi_w, pi_b, interpret=False):
    bg = (b_ih + b_hh)[None, :]

    af, ab = _edge_mats_sc(temporal_edge_i[:, 0, :], temporal_edge_i[:, 1, :],
                           temporal_edge_w)

    xin = pl.pallas_call(
        _graph_kernel,
        grid=(B // G,),
        in_specs=[
            pl.BlockSpec((G, T, F), lambda b: (b, 0, 0)),
            pl.BlockSpec((G, T, T), lambda b: (b, 0, 0)),
            pl.BlockSpec((G, T, T), lambda b: (b, 0, 0)),
            pl.BlockSpec((2, E_S), lambda b: (0, 0)),
            pl.BlockSpec((1, E_S), lambda b: (0, 0)),
            pl.BlockSpec((4 * F, H), lambda b: (0, 0)),
            pl.BlockSpec((1, H), lambda b: (0, 0)),
            pl.BlockSpec((4, T, T), lambda b: (0, 0, 0)),
            pl.BlockSpec((T, 1), lambda b: (0, 0)),
        ],
        out_specs=pl.BlockSpec((G, T, XIN), lambda b: (b, 0, 0)),
        out_shape=jax.ShapeDtypeStruct((B, T, XIN), _BF16),
        scratch_shapes=[pltpu.VMEM((F, F), _BF16), pltpu.VMEM((F, F), _BF16)],
        interpret=interpret,
    )(x, af.reshape(B, T, T), ab.reshape(B, T, T), edge_index,
      edge_weight[None, :], _b16(Wt), bt[None, :],
      _b16(jnp.swapaxes(Ws.reshape(4, T, T), 1, 2)), bs[:, None])

    mu_f, sg_f, pi_f = pl.pallas_call(
        _proj_lstm_head_kernel,
        out_shape=[
            jax.ShapeDtypeStruct((B * T, M * OUT), _F32),
            jax.ShapeDtypeStruct((B * T, M * OUT), _F32),
            jax.ShapeDtypeStruct((B * T, M), _F32),
        ],
        scratch_shapes=[pltpu.VMEM((T, B, 4 * H), _F32),
                        pltpu.VMEM((T, B, H), _F32)],
        interpret=interpret,
    )(jnp.swapaxes(xin, 0, 1), _b16(W_ih.T), bg, _b16(W_hh.T),
      _b16(mu_w.T), mu_b[None, :], _b16(sigma_w.T), sigma_b[None, :],
      _b16(pi_w.T), pi_b[None, :])

    mu = mu_f.reshape(B, T, M, OUT)
    sigma = sg_f.reshape(B, T, M, OUT)
    pi = pi_f.reshape(B, T, M)
    return mu, sigma, pi


# speedup vs baseline: 1.0406x; 1.0406x over previous
import functools

import jax
import jax.numpy as jnp
from jax.experimental import pallas as pl
from jax.experimental.pallas import tpu as pltpu
from jax.experimental.pallas import tpu_sc as plsc

B = 32
T = 64
F = 128
H = 256
M = 5
OUT = 128
E_T = 512
E_S = 128
XIN = H + 2 * F
G = 8

_F32 = jnp.float32
_BF16 = jnp.bfloat16


def _dot(a, b):
    return jax.lax.dot_general(a, b, (((1,), (0,)), ((), ())),
                               preferred_element_type=_F32)


def _dot_t(a, b):
    return jax.lax.dot_general(a, b, (((1,), (1,)), ((), ())),
                               preferred_element_type=_F32)


def _dot_lt(a, b):
    return jax.lax.dot_general(a, b, (((0,), (0,)), ((), ())),
                               preferred_element_type=_F32)


def _b16(x):
    return x.astype(_BF16)


_LN = 16


def _edge_mats_sc_body(src_hbm, dst_hbm, wgt_hbm, af_hbm, ab_hbm,
                       src_v, dst_v, w_v, dego_v, degi_v, afv, abv):
    c = jax.lax.axis_index("c")
    s = jax.lax.axis_index("s")
    wid = s * 2 + c
    pltpu.sync_copy(src_hbm.at[wid], src_v)
    pltpu.sync_copy(dst_hbm.at[wid], dst_v)
    pltpu.sync_copy(wgt_hbm.at[wid], w_v)
    zf = jnp.zeros((_LN,), _F32)
    for i in range(T // _LN):
        dego_v[pl.ds(i * _LN, _LN)] = zf
        degi_v[pl.ds(i * _LN, _LN)] = zf
    for i in range(E_T // _LN):
        sl = pl.ds(i * _LN, _LN)
        plsc.addupdate_scatter(dego_v, [src_v[sl]], w_v[sl])
        plsc.addupdate_scatter(degi_v, [dst_v[sl]], w_v[sl])
    for i in range(T * T // _LN):
        sl = pl.ds(i * _LN, _LN)
        afv[sl] = zf
        abv[sl] = zf
    for i in range(T // _LN):
        sl = pl.ds(i * _LN, _LN)
        d = dego_v[sl]
        dego_v[sl] = jnp.where(d > 0, d, 1.0)
        d = degi_v[sl]
        degi_v[sl] = jnp.where(d > 0, d, 1.0)
    for i in range(E_T // _LN):
        sl = pl.ds(i * _LN, _LN)
        sv = src_v[sl]
        dv = dst_v[sl]
        wv = w_v[sl]
        wf = wv / plsc.load_gather(dego_v, [sv])
        wb = wv / plsc.load_gather(degi_v, [dv])
        plsc.addupdate_scatter(afv, [dv * T + sv], wf)
        plsc.addupdate_scatter(abv, [sv * T + dv], wb)
    pltpu.sync_copy(afv, af_hbm.at[wid])
    pltpu.sync_copy(abv, ab_hbm.at[wid])


def _edge_mats_sc(srcs, dsts, wgts):
    k = functools.partial(
        pl.kernel,
        out_type=[jax.ShapeDtypeStruct((B, T * T), _F32),
                  jax.ShapeDtypeStruct((B, T * T), _F32)],
        mesh=plsc.VectorSubcoreMesh(core_axis_name="c", subcore_axis_name="s"),
        compiler_params=pltpu.CompilerParams(needs_layout_passes=False),
        scratch_types=[pltpu.VMEM((E_T,), jnp.int32),
                       pltpu.VMEM((E_T,), jnp.int32),
                       pltpu.VMEM((E_T,), _F32),
                       pltpu.VMEM((T,), _F32),
                       pltpu.VMEM((T,), _F32),
                       pltpu.VMEM((T * T,), _F32),
                       pltpu.VMEM((T * T,), _F32)],
    )(_edge_mats_sc_body)
    return k(srcs, dsts, wgts)


NP = B // G


def _model_kernel(x_ref, af_ref, ab_ref, ei_ref, ew_ref,
                  wt_ref, bt_ref, ws_ref, bs_ref,
                  wih_ref, bg_ref, whh_ref,
                  muw_ref, mub_ref, sgw_ref, sgb_ref, piw_ref, pib_ref,
                  mu_ref, sg_ref, pi_ref,
                  xin_ref, afs_ref, abs_ref, p_ref, hs_ref):
    b = pl.program_id(0)

    @pl.when(b == 0)
    def _():
        src = ei_ref[0:1, :].astype(jnp.int32)
        dst = ei_ref[1:2, :].astype(jnp.int32)
        w = ew_ref[...]
        iota = jax.lax.broadcasted_iota(jnp.int32, (F, E_S), 0)
        gs = (iota == src).astype(_F32)
        gd = (iota == dst).astype(_F32)
        deg_out = jnp.sum(gs * w, axis=1, keepdims=True)
        deg_in = jnp.sum(gd * w, axis=1, keepdims=True)
        dso = jnp.where(deg_out > 0, deg_out, 1.0)
        dsi = jnp.where(deg_in > 0, deg_in, 1.0)
        w_fwd = w / jnp.sum(gs * dso, axis=0, keepdims=True)
        w_bwd = w / jnp.sum(gd * dsi, axis=0, keepdims=True)
        afs_ref[...] = _b16(_dot_t(_b16(gs), _b16(gd * w_fwd)))
        abs_ref[...] = _b16(_dot_t(_b16(gd), _b16(gs * w_bwd)))

    @pl.when(b < NP)
    def _():
        afs = afs_ref[...]
        abs_ = abs_ref[...]
        for j in range(G):
            a_f = _b16(af_ref[j])
            a_b = _b16(ab_ref[j])

            xb = _b16(x_ref[j])
            zf1 = _b16(_dot(a_f, xb))
            zf2 = _b16(_dot(a_f, zf1))
            zb1 = _b16(_dot(a_b, xb))
            zb2 = _b16(_dot(a_b, zb1))
            dt = (_dot(zf1, wt_ref[0:F]) + _dot(zf2, wt_ref[F:2 * F])
                  + _dot(zb1, wt_ref[2 * F:3 * F])
                  + _dot(zb2, wt_ref[3 * F:4 * F])
                  + bt_ref[...])

            y1 = _b16(_dot(xb, afs))
            y2 = _b16(_dot(y1, afs))
            y3 = _b16(_dot(xb, abs_))
            y4 = _b16(_dot(y3, abs_))
            ds = (_dot(ws_ref[0], y1) + _dot(ws_ref[1], y2)
                  + _dot(ws_ref[2], y3) + _dot(ws_ref[3], y4)
                  + bs_ref[...])

            row = jnp.concatenate([dt, ds, x_ref[j]], axis=1)
            xin_ref[:, pl.ds(b * G + j, 1), :] = row.reshape(T, 1, XIN)

    @pl.when(b == NP)
    def _():
        xin = _b16(xin_ref[...].reshape(T * B, XIN))
        p_ref[...] = (_dot(xin, wih_ref[...])
                      + bg_ref[...]).reshape(T, B, 4 * H)

        def _sig(v):
            return 0.5 * jnp.tanh(0.5 * v) + 0.5

        def step(t, carry):
            h, c = carry
            pt = p_ref[pl.ds(t, 1)].reshape(B, 4 * H)
            gates = pt + _dot(h, whh_ref[...])
            i = _sig(gates[:, 0:H])
            f = _sig(gates[:, H:2 * H])
            g = jnp.tanh(gates[:, 2 * H:3 * H])
            o = _sig(gates[:, 3 * H:4 * H])
            c2 = f * c + i * g
            h2 = o * jnp.tanh(c2)
            hs_ref[pl.ds(t, 1)] = h2.reshape(1, B, H)
            return (_b16(h2), c2)

        zeros16 = jnp.zeros((B, H), _BF16)
        zeros = jnp.zeros((B, H), _F32)
        jax.lax.fori_loop(0, T, step, (zeros16, zeros))

        hs = _b16(jnp.swapaxes(hs_ref[...], 0, 1).reshape(B * T, H))
        mu_ref[...] = _dot(hs, muw_ref[...]) + mub_ref[...]
        sg_ref[...] = jnp.exp(_dot(hs, sgw_ref[...]) + sgb_ref[...])
        logits = _dot(hs, piw_ref[...]) + pib_ref[...]
        mx = jnp.max(logits, axis=-1, keepdims=True)
        e = jnp.exp(logits - mx)
        pi_ref[...] = e / jnp.sum(e, axis=-1, keepdims=True)


def kernel(x, temporal_edge_i, temporal_edge_w, edge_index, edge_weight,
           Wt, bt, Ws, bs, W_ih, W_hh, b_ih, b_hh,
           mu_w, mu_b, sigma_w, sigma_b, pi_w, pi_b, interpret=False):
    bg = (b_ih + b_hh)[None, :]

    af, ab = _edge_mats_sc(temporal_edge_i[:, 0, :], temporal_edge_i[:, 1, :],
                           temporal_edge_w)

    last = B // G - 1
    sample_ix = lambda b: (jnp.minimum(b, last), 0, 0)
    const2 = lambda b: (0, 0)
    const3 = lambda b: (0, 0, 0)

    mu_f, sg_f, pi_f = pl.pallas_call(
        _model_kernel,
        grid=(NP + 1,),
        in_specs=[
            pl.BlockSpec((G, T, F), sample_ix),
            pl.BlockSpec((G, T, T), sample_ix),
            pl.BlockSpec((G, T, T), sample_ix),
            pl.BlockSpec((2, E_S), const2),
            pl.BlockSpec((1, E_S), const2),
            pl.BlockSpec((4 * F, H), const2),
            pl.BlockSpec((1, H), const2),
            pl.BlockSpec((4, T, T), const3),
            pl.BlockSpec((T, 1), const2),
            pl.BlockSpec((XIN, 4 * H), const2),
            pl.BlockSpec((1, 4 * H), const2),
            pl.BlockSpec((H, 4 * H), const2),
            pl.BlockSpec((H, M * OUT), const2),
            pl.BlockSpec((1, M * OUT), const2),
            pl.BlockSpec((H, M * OUT), const2),
            pl.BlockSpec((1, M * OUT), const2),
            pl.BlockSpec((H, M), const2),
            pl.BlockSpec((1, M), const2),
        ],
        out_specs=[
            pl.BlockSpec((B * T, M * OUT), const2),
            pl.BlockSpec((B * T, M * OUT), const2),
            pl.BlockSpec((B * T, M), const2),
        ],
        out_shape=[
            jax.ShapeDtypeStruct((B * T, M * OUT), _F32),
            jax.ShapeDtypeStruct((B * T, M * OUT), _F32),
            jax.ShapeDtypeStruct((B * T, M), _F32),
        ],
        scratch_shapes=[pltpu.VMEM((T, B, XIN), _F32),
                        pltpu.VMEM((F, F), _BF16),
                        pltpu.VMEM((F, F), _BF16),
                        pltpu.VMEM((T, B, 4 * H), _F32),
                        pltpu.VMEM((T, B, H), _F32)],
        interpret=interpret,
    )(x, af.reshape(B, T, T), ab.reshape(B, T, T), edge_index,
      edge_weight[None, :], _b16(Wt), bt[None, :],
      _b16(jnp.swapaxes(Ws.reshape(4, T, T), 1, 2)), bs[:, None],
      _b16(W_ih.T), bg, _b16(W_hh.T),
      _b16(mu_w.T), mu_b[None, :], _b16(sigma_w.T), sigma_b[None, :],
      _b16(pi_w.T), pi_b[None, :])

    mu = mu_f.reshape(B, T, M, OUT)
    sigma = sg_f.reshape(B, T, M, OUT)
    pi = pi_f.reshape(B, T, M)
    return mu, sigma, pi


# speedup vs baseline: 1.0436x; 1.0028x over previous
import functools

import jax
import jax.numpy as jnp
from jax.experimental import pallas as pl
from jax.experimental.pallas import tpu as pltpu
from jax.experimental.pallas import tpu_sc as plsc

B = 32
T = 64
F = 128
H = 256
M = 5
OUT = 128
E_T = 512
E_S = 128
XIN = H + 2 * F
G = 16

_F32 = jnp.float32
_BF16 = jnp.bfloat16


def _dot(a, b):
    return jax.lax.dot_general(a, b, (((1,), (0,)), ((), ())),
                               preferred_element_type=_F32)


def _dot_t(a, b):
    return jax.lax.dot_general(a, b, (((1,), (1,)), ((), ())),
                               preferred_element_type=_F32)


def _dot_lt(a, b):
    return jax.lax.dot_general(a, b, (((0,), (0,)), ((), ())),
                               preferred_element_type=_F32)


def _b16(x):
    return x.astype(_BF16)


_LN = 16


def _edge_mats_sc_body(src_hbm, dst_hbm, wgt_hbm, af_hbm, ab_hbm,
                       src_v, dst_v, w_v, dego_v, degi_v, afv, abv):
    c = jax.lax.axis_index("c")
    s = jax.lax.axis_index("s")
    wid = s * 2 + c
    pltpu.sync_copy(src_hbm.at[wid], src_v)
    pltpu.sync_copy(dst_hbm.at[wid], dst_v)
    pltpu.sync_copy(wgt_hbm.at[wid], w_v)
    zf = jnp.zeros((_LN,), _F32)
    for i in range(T // _LN):
        dego_v[pl.ds(i * _LN, _LN)] = zf
        degi_v[pl.ds(i * _LN, _LN)] = zf
    for i in range(E_T // _LN):
        sl = pl.ds(i * _LN, _LN)
        plsc.addupdate_scatter(dego_v, [src_v[sl]], w_v[sl])
        plsc.addupdate_scatter(degi_v, [dst_v[sl]], w_v[sl])
    for i in range(T * T // _LN):
        sl = pl.ds(i * _LN, _LN)
        afv[sl] = zf
        abv[sl] = zf
    for i in range(T // _LN):
        sl = pl.ds(i * _LN, _LN)
        d = dego_v[sl]
        dego_v[sl] = jnp.where(d > 0, d, 1.0)
        d = degi_v[sl]
        degi_v[sl] = jnp.where(d > 0, d, 1.0)
    for i in range(E_T // _LN):
        sl = pl.ds(i * _LN, _LN)
        sv = src_v[sl]
        dv = dst_v[sl]
        wv = w_v[sl]
        wf = wv / plsc.load_gather(dego_v, [sv])
        wb = wv / plsc.load_gather(degi_v, [dv])
        plsc.addupdate_scatter(afv, [dv * T + sv], wf)
        plsc.addupdate_scatter(abv, [sv * T + dv], wb)
    pltpu.sync_copy(afv, af_hbm.at[wid])
    pltpu.sync_copy(abv, ab_hbm.at[wid])


def _edge_mats_sc(srcs, dsts, wgts):
    k = functools.partial(
        pl.kernel,
        out_type=[jax.ShapeDtypeStruct((B, T * T), _F32),
                  jax.ShapeDtypeStruct((B, T * T), _F32)],
        mesh=plsc.VectorSubcoreMesh(core_axis_name="c", subcore_axis_name="s"),
        compiler_params=pltpu.CompilerParams(needs_layout_passes=False),
        scratch_types=[pltpu.VMEM((E_T,), jnp.int32),
                       pltpu.VMEM((E_T,), jnp.int32),
                       pltpu.VMEM((E_T,), _F32),
                       pltpu.VMEM((T,), _F32),
                       pltpu.VMEM((T,), _F32),
                       pltpu.VMEM((T * T,), _F32),
                       pltpu.VMEM((T * T,), _F32)],
    )(_edge_mats_sc_body)
    return k(srcs, dsts, wgts)


NP = B // G


def _model_kernel(x_ref, af_ref, ab_ref, ei_ref, ew_ref,
                  wt_ref, bt_ref, ws_ref, bs_ref,
                  wih_ref, bg_ref, whh_ref,
                  muw_ref, mub_ref, sgw_ref, sgb_ref, piw_ref, pib_ref,
                  mu_ref, sg_ref, pi_ref,
                  xin_ref, afs_ref, abs_ref, p_ref, hs_ref):
    b = pl.program_id(0)

    @pl.when(b == 0)
    def _():
        src = ei_ref[0:1, :].astype(jnp.int32)
        dst = ei_ref[1:2, :].astype(jnp.int32)
        w = ew_ref[...]
        iota = jax.lax.broadcasted_iota(jnp.int32, (F, E_S), 0)
        gs = (iota == src).astype(_F32)
        gd = (iota == dst).astype(_F32)
        deg_out = jnp.sum(gs * w, axis=1, keepdims=True)
        deg_in = jnp.sum(gd * w, axis=1, keepdims=True)
        dso = jnp.where(deg_out > 0, deg_out, 1.0)
        dsi = jnp.where(deg_in > 0, deg_in, 1.0)
        w_fwd = w / jnp.sum(gs * dso, axis=0, keepdims=True)
        w_bwd = w / jnp.sum(gd * dsi, axis=0, keepdims=True)
        afs_ref[...] = _b16(_dot_t(_b16(gs), _b16(gd * w_fwd)))
        abs_ref[...] = _b16(_dot_t(_b16(gd), _b16(gs * w_bwd)))

    @pl.when(b < NP)
    def _():
        afs = afs_ref[...]
        abs_ = abs_ref[...]
        for j in range(G):
            a_f = _b16(af_ref[j])
            a_b = _b16(ab_ref[j])

            xb = _b16(x_ref[j])
            zf1 = _b16(_dot(a_f, xb))
            zf2 = _b16(_dot(a_f, zf1))
            zb1 = _b16(_dot(a_b, xb))
            zb2 = _b16(_dot(a_b, zb1))
            dt = (_dot(zf1, wt_ref[0:F]) + _dot(zf2, wt_ref[F:2 * F])
                  + _dot(zb1, wt_ref[2 * F:3 * F])
                  + _dot(zb2, wt_ref[3 * F:4 * F])
                  + bt_ref[...])

            y1 = _b16(_dot(xb, afs))
            y2 = _b16(_dot(y1, afs))
            y3 = _b16(_dot(xb, abs_))
            y4 = _b16(_dot(y3, abs_))
            ds = (_dot(ws_ref[0], y1) + _dot(ws_ref[1], y2)
                  + _dot(ws_ref[2], y3) + _dot(ws_ref[3], y4)
                  + bs_ref[...])

            row = jnp.concatenate([dt, ds, x_ref[j]], axis=1)
            xin_ref[:, pl.ds(b * G + j, 1), :] = row.reshape(T, 1, XIN)

    @pl.when(b == NP)
    def _():
        xin = _b16(xin_ref[...].reshape(T * B, XIN))
        p_ref[...] = (_dot(xin, wih_ref[...])
                      + bg_ref[...]).reshape(T, B, 4 * H)

        def _sig(v):
            return 0.5 * jnp.tanh(0.5 * v) + 0.5

        def step(t, carry):
            h, c = carry
            pt = p_ref[pl.ds(t, 1)].reshape(B, 4 * H)
            gates = pt + _dot(h, whh_ref[...])
            i = _sig(gates[:, 0:H])
            f = _sig(gates[:, H:2 * H])
            g = jnp.tanh(gates[:, 2 * H:3 * H])
            o = _sig(gates[:, 3 * H:4 * H])
            c2 = f * c + i * g
            h2 = o * jnp.tanh(c2)
            hs_ref[pl.ds(t, 1)] = h2.reshape(1, B, H)
            return (_b16(h2), c2)

        zeros16 = jnp.zeros((B, H), _BF16)
        zeros = jnp.zeros((B, H), _F32)
        jax.lax.fori_loop(0, T, step, (zeros16, zeros))

        hs = _b16(jnp.swapaxes(hs_ref[...], 0, 1).reshape(B * T, H))
        mu_ref[...] = _dot(hs, muw_ref[...]) + mub_ref[...]
        sg_ref[...] = jnp.exp(_dot(hs, sgw_ref[...]) + sgb_ref[...])
        logits = _dot(hs, piw_ref[...]) + pib_ref[...]
        mx = jnp.max(logits, axis=-1, keepdims=True)
        e = jnp.exp(logits - mx)
        pi_ref[...] = e / jnp.sum(e, axis=-1, keepdims=True)


def kernel(x, temporal_edge_i, temporal_edge_w, edge_index, edge_weight,
           Wt, bt, Ws, bs, W_ih, W_hh, b_ih, b_hh,
           mu_w, mu_b, sigma_w, sigma_b, pi_w, pi_b, interpret=False):
    bg = (b_ih + b_hh)[None, :]

    af, ab = _edge_mats_sc(temporal_edge_i[:, 0, :], temporal_edge_i[:, 1, :],
                           temporal_edge_w)

    last = B // G - 1
    sample_ix = lambda b: (jnp.minimum(b, last), 0, 0)
    const2 = lambda b: (0, 0)
    const3 = lambda b: (0, 0, 0)

    mu_f, sg_f, pi_f = pl.pallas_call(
        _model_kernel,
        grid=(NP + 1,),
        in_specs=[
            pl.BlockSpec((G, T, F), sample_ix),
            pl.BlockSpec((G, T, T), sample_ix),
            pl.BlockSpec((G, T, T), sample_ix),
            pl.BlockSpec((2, E_S), const2),
            pl.BlockSpec((1, E_S), const2),
            pl.BlockSpec((4 * F, H), const2),
            pl.BlockSpec((1, H), const2),
            pl.BlockSpec((4, T, T), const3),
            pl.BlockSpec((T, 1), const2),
            pl.BlockSpec((XIN, 4 * H), const2),
            pl.BlockSpec((1, 4 * H), const2),
            pl.BlockSpec((H, 4 * H), const2),
            pl.BlockSpec((H, M * OUT), const2),
            pl.BlockSpec((1, M * OUT), const2),
            pl.BlockSpec((H, M * OUT), const2),
            pl.BlockSpec((1, M * OUT), const2),
            pl.BlockSpec((H, M), const2),
            pl.BlockSpec((1, M), const2),
        ],
        out_specs=[
            pl.BlockSpec((B * T, M * OUT), const2),
            pl.BlockSpec((B * T, M * OUT), const2),
            pl.BlockSpec((B * T, M), const2),
        ],
        out_shape=[
            jax.ShapeDtypeStruct((B * T, M * OUT), _F32),
            jax.ShapeDtypeStruct((B * T, M * OUT), _F32),
            jax.ShapeDtypeStruct((B * T, M), _F32),
        ],
        scratch_shapes=[pltpu.VMEM((T, B, XIN), _F32),
                        pltpu.VMEM((F, F), _BF16),
                        pltpu.VMEM((F, F), _BF16),
                        pltpu.VMEM((T, B, 4 * H), _F32),
                        pltpu.VMEM((T, B, H), _F32)],
        interpret=interpret,
    )(x, af.reshape(B, T, T), ab.reshape(B, T, T), edge_index,
      edge_weight[None, :], _b16(Wt), bt[None, :],
      _b16(jnp.swapaxes(Ws.reshape(4, T, T), 1, 2)), bs[:, None],
      _b16(W_ih.T), bg, _b16(W_hh.T),
      _b16(mu_w.T), mu_b[None, :], _b16(sigma_w.T), sigma_b[None, :],
      _b16(pi_w.T), pi_b[None, :])

    mu = mu_f.reshape(B, T, M, OUT)
    sigma = sg_f.reshape(B, T, M, OUT)
    pi = pi_f.reshape(B, T, M)
    return mu, sigma, pi


# speedup vs baseline: 1.0812x; 1.0361x over previous
import functools

import jax
import jax.numpy as jnp
from jax.experimental import pallas as pl
from jax.experimental.pallas import tpu as pltpu
from jax.experimental.pallas import tpu_sc as plsc

B = 32
T = 64
F = 128
H = 256
M = 5
OUT = 128
E_T = 512
E_S = 128
XIN = H + 2 * F
G = 16

_F32 = jnp.float32
_BF16 = jnp.bfloat16


def _dot(a, b):
    return jax.lax.dot_general(a, b, (((1,), (0,)), ((), ())),
                               preferred_element_type=_F32)


def _dot_t(a, b):
    return jax.lax.dot_general(a, b, (((1,), (1,)), ((), ())),
                               preferred_element_type=_F32)


def _dot_lt(a, b):
    return jax.lax.dot_general(a, b, (((0,), (0,)), ((), ())),
                               preferred_element_type=_F32)


def _b16(x):
    return x.astype(_BF16)


_LN = 16


def _edge_mats_sc_body(tei_hbm, wgt_hbm, af_hbm, ab_hbm,
                       src_v, dst_v, w_v, dego_v, degi_v, afv, abv):
    c = jax.lax.axis_index("c")
    s = jax.lax.axis_index("s")
    wid = s * 2 + c
    pltpu.sync_copy(tei_hbm.at[wid, 0], src_v)
    pltpu.sync_copy(tei_hbm.at[wid, 1], dst_v)
    pltpu.sync_copy(wgt_hbm.at[wid], w_v)
    zf = jnp.zeros((_LN,), _F32)
    for i in range(T // _LN):
        dego_v[pl.ds(i * _LN, _LN)] = zf
        degi_v[pl.ds(i * _LN, _LN)] = zf
    for i in range(E_T // _LN):
        sl = pl.ds(i * _LN, _LN)
        plsc.addupdate_scatter(dego_v, [src_v[sl]], w_v[sl])
        plsc.addupdate_scatter(degi_v, [dst_v[sl]], w_v[sl])
    for i in range(T * T // _LN):
        sl = pl.ds(i * _LN, _LN)
        afv[sl] = zf
        abv[sl] = zf
    for i in range(T // _LN):
        sl = pl.ds(i * _LN, _LN)
        d = dego_v[sl]
        dego_v[sl] = jnp.where(d > 0, d, 1.0)
        d = degi_v[sl]
        degi_v[sl] = jnp.where(d > 0, d, 1.0)
    for i in range(E_T // _LN):
        sl = pl.ds(i * _LN, _LN)
        sv = src_v[sl]
        dv = dst_v[sl]
        wv = w_v[sl]
        wf = wv / plsc.load_gather(dego_v, [sv])
        wb = wv / plsc.load_gather(degi_v, [dv])
        plsc.addupdate_scatter(afv, [dv * T + sv], wf)
        plsc.addupdate_scatter(abv, [sv * T + dv], wb)
    pltpu.sync_copy(afv, af_hbm.at[wid])
    pltpu.sync_copy(abv, ab_hbm.at[wid])


def _edge_mats_sc(tei, wgts):
    k = functools.partial(
        pl.kernel,
        out_type=[jax.ShapeDtypeStruct((B, T * T), _F32),
                  jax.ShapeDtypeStruct((B, T * T), _F32)],
        mesh=plsc.VectorSubcoreMesh(core_axis_name="c", subcore_axis_name="s"),
        compiler_params=pltpu.CompilerParams(needs_layout_passes=False),
        scratch_types=[pltpu.VMEM((E_T,), jnp.int32),
                       pltpu.VMEM((E_T,), jnp.int32),
                       pltpu.VMEM((E_T,), _F32),
                       pltpu.VMEM((T,), _F32),
                       pltpu.VMEM((T,), _F32),
                       pltpu.VMEM((T * T,), _F32),
                       pltpu.VMEM((T * T,), _F32)],
    )(_edge_mats_sc_body)
    return k(tei, wgts)


NP = B // G


def _model_kernel(x_ref, af_ref, ab_ref, ei_ref, ew_ref,
                  wt_ref, bt_ref, ws_ref, bs_ref,
                  wih_ref, bih_ref, bhh_ref, whh_ref,
                  muw_ref, mub_ref, sgw_ref, sgb_ref, piw_ref, pib_ref,
                  mu_ref, sg_ref, pi_ref,
                  xin_ref, afs_ref, abs_ref, p_ref, hs_ref,
                  wt16_ref, ws16_ref, whh16_ref):
    b = pl.program_id(0)

    @pl.when(b == 0)
    def _():
        wt16_ref[...] = _b16(wt_ref[...])
        ws16_ref[...] = _b16(ws_ref[...])
        whh16_ref[...] = _b16(whh_ref[...])
        src = ei_ref[0:1, :].astype(jnp.int32)
        dst = ei_ref[1:2, :].astype(jnp.int32)
        w = ew_ref[...]
        iota = jax.lax.broadcasted_iota(jnp.int32, (F, E_S), 0)
        gs = (iota == src).astype(_F32)
        gd = (iota == dst).astype(_F32)
        deg_out = jnp.sum(gs * w, axis=1, keepdims=True)
        deg_in = jnp.sum(gd * w, axis=1, keepdims=True)
        dso = jnp.where(deg_out > 0, deg_out, 1.0)
        dsi = jnp.where(deg_in > 0, deg_in, 1.0)
        w_fwd = w / jnp.sum(gs * dso, axis=0, keepdims=True)
        w_bwd = w / jnp.sum(gd * dsi, axis=0, keepdims=True)
        afs_ref[...] = _b16(_dot_t(_b16(gs), _b16(gd * w_fwd)))
        abs_ref[...] = _b16(_dot_t(_b16(gd), _b16(gs * w_bwd)))

    @pl.when(b < NP)
    def _():
        afs = afs_ref[...]
        abs_ = abs_ref[...]
        for j in range(G):
            a_f = _b16(af_ref[j])
            a_b = _b16(ab_ref[j])

            xb = _b16(x_ref[j])
            zf1 = _b16(_dot(a_f, xb))
            zf2 = _b16(_dot(a_f, zf1))
            zb1 = _b16(_dot(a_b, xb))
            zb2 = _b16(_dot(a_b, zb1))
            dt = (_dot(zf1, wt16_ref[0:F]) + _dot(zf2, wt16_ref[F:2 * F])
                  + _dot(zb1, wt16_ref[2 * F:3 * F])
                  + _dot(zb2, wt16_ref[3 * F:4 * F])
                  + bt_ref[...])

            y1 = _b16(_dot(xb, afs))
            y2 = _b16(_dot(y1, afs))
            y3 = _b16(_dot(xb, abs_))
            y4 = _b16(_dot(y3, abs_))
            ds = (_dot_lt(ws16_ref[0:T], y1) + _dot_lt(ws16_ref[T:2 * T], y2)
                  + _dot_lt(ws16_ref[2 * T:3 * T], y3)
                  + _dot_lt(ws16_ref[3 * T:4 * T], y4)
                  + bs_ref[...])

            row = jnp.concatenate([dt, ds, x_ref[j]], axis=1)
            xin_ref[:, pl.ds(b * G + j, 1), :] = row.reshape(T, 1, XIN)

    @pl.when(b == NP)
    def _():
        xin = _b16(xin_ref[...].reshape(T * B, XIN))
        bg = bih_ref[...] + bhh_ref[...]
        p_ref[...] = (_dot_t(xin, _b16(wih_ref[...]))
                      + bg).reshape(T, B, 4 * H)

        def _sig(v):
            return 0.5 * jnp.tanh(0.5 * v) + 0.5

        def step(t, carry):
            h, c = carry
            pt = p_ref[pl.ds(t, 1)].reshape(B, 4 * H)
            gates = pt + _dot_t(h, whh16_ref[...])
            i = _sig(gates[:, 0:H])
            f = _sig(gates[:, H:2 * H])
            g = jnp.tanh(gates[:, 2 * H:3 * H])
            o = _sig(gates[:, 3 * H:4 * H])
            c2 = f * c + i * g
            h2 = o * jnp.tanh(c2)
            hs_ref[pl.ds(t, 1)] = h2.reshape(1, B, H)
            return (_b16(h2), c2)

        zeros16 = jnp.zeros((B, H), _BF16)
        zeros = jnp.zeros((B, H), _F32)
        jax.lax.fori_loop(0, T, step, (zeros16, zeros))

        hs = _b16(jnp.swapaxes(hs_ref[...], 0, 1).reshape(B * T, H))
        mu_ref[...] = _dot_t(hs, _b16(muw_ref[...])) + mub_ref[...]
        sg_ref[...] = jnp.exp(_dot_t(hs, _b16(sgw_ref[...])) + sgb_ref[...])
        logits = _dot_t(hs, _b16(piw_ref[...])) + pib_ref[...]
        mx = jnp.max(logits, axis=-1, keepdims=True)
        e = jnp.exp(logits - mx)
        pi_ref[...] = e / jnp.sum(e, axis=-1, keepdims=True)


def kernel(x, temporal_edge_i, temporal_edge_w, edge_index, edge_weight,
           Wt, bt, Ws, bs, W_ih, W_hh, b_ih, b_hh,
           mu_w, mu_b, sigma_w, sigma_b, pi_w, pi_b, interpret=False):
    af, ab = _edge_mats_sc(temporal_edge_i, temporal_edge_w)

    last = B // G - 1
    sample_ix = lambda b: (jnp.minimum(b, last), 0, 0)
    const2 = lambda b: (0, 0)
    const3 = lambda b: (0, 0, 0)

    mu_f, sg_f, pi_f = pl.pallas_call(
        _model_kernel,
        grid=(NP + 1,),
        in_specs=[
            pl.BlockSpec((G, T, F), sample_ix),
            pl.BlockSpec((G, T, T), sample_ix),
            pl.BlockSpec((G, T, T), sample_ix),
            pl.BlockSpec((2, E_S), const2),
            pl.BlockSpec((1, E_S), const2),
            pl.BlockSpec((4 * F, H), const2),
            pl.BlockSpec((1, H), const2),
            pl.BlockSpec((4 * T, T), const2),
            pl.BlockSpec((T, 1), const2),
            pl.BlockSpec((4 * H, XIN), const2),
            pl.BlockSpec((1, 4 * H), const2),
            pl.BlockSpec((1, 4 * H), const2),
            pl.BlockSpec((4 * H, H), const2),
            pl.BlockSpec((M * OUT, H), const2),
            pl.BlockSpec((1, M * OUT), const2),
            pl.BlockSpec((M * OUT, H), const2),
            pl.BlockSpec((1, M * OUT), const2),
            pl.BlockSpec((M, H), const2),
            pl.BlockSpec((1, M), const2),
        ],
        out_specs=[
            pl.BlockSpec((B * T, M * OUT), const2),
            pl.BlockSpec((B * T, M * OUT), const2),
            pl.BlockSpec((B * T, M), const2),
        ],
        out_shape=[
            jax.ShapeDtypeStruct((B * T, M * OUT), _F32),
            jax.ShapeDtypeStruct((B * T, M * OUT), _F32),
            jax.ShapeDtypeStruct((B * T, M), _F32),
        ],
        scratch_shapes=[pltpu.VMEM((T, B, XIN), _F32),
                        pltpu.VMEM((F, F), _BF16),
                        pltpu.VMEM((F, F), _BF16),
                        pltpu.VMEM((T, B, 4 * H), _F32),
                        pltpu.VMEM((T, B, H), _F32),
                        pltpu.VMEM((4 * F, H), _BF16),
                        pltpu.VMEM((4 * T, T), _BF16),
                        pltpu.VMEM((4 * H, H), _BF16)],
        interpret=interpret,
    )(x, af.reshape(B, T, T), ab.reshape(B, T, T), edge_index,
      edge_weight[None, :], Wt, bt[None, :], Ws, bs[:, None],
      W_ih, b_ih[None, :], b_hh[None, :], W_hh,
      mu_w, mu_b[None, :], sigma_w, sigma_b[None, :],
      pi_w, pi_b[None, :])

    mu = mu_f.reshape(B, T, M, OUT)
    sigma = sg_f.reshape(B, T, M, OUT)
    pi = pi_f.reshape(B, T, M)
    return mu, sigma, pi
